# baseline (device time: 26279 ns/iter reference)
import jax
import jax.numpy as jnp
from jax import lax
from jax.experimental import pallas as pl
from jax.experimental.pallas import tpu as pltpu

N_DEV = 4


def kernel(x, w_mat):
    m_per, k = x.shape
    n = w_mat.shape[1]
    h_half = m_per // 2
    h_q = h_half // 2

    def body(x_ref, w_ref, out_ref, comm_ref, send_sems, recv_sems):
        my_pos = lax.axis_index("i")
        left = (my_pos - 1) % N_DEV
        right = (my_pos + 1) % N_DEV
        opp = (my_pos + 2) % N_DEV

        def copy(t, src, dst, target):
            return pltpu.make_async_remote_copy(
                src_ref=src,
                dst_ref=dst,
                send_sem=send_sems.at[t],
                recv_sem=recv_sems.at[t],
                device_id=(target,),
                device_id_type=pl.DeviceIdType.MESH,
            )

        x_a = x_ref.at[pl.ds(0, h_half)]
        x_b = x_ref.at[pl.ds(h_half, h_half)]
        f1a = copy(0, x_a, comm_ref.at[0], right)
        f1b = copy(1, x_b, comm_ref.at[1], right)
        b1b = copy(2, x_b, comm_ref.at[3], left)
        b1a = copy(3, x_a, comm_ref.at[2], left)
        f2q0 = copy(4, comm_ref.at[0, pl.ds(0, h_q)],
                    comm_ref.at[4, pl.ds(0, h_q)], right)
        f2q1 = copy(5, comm_ref.at[0, pl.ds(h_q, h_q)],
                    comm_ref.at[4, pl.ds(h_q, h_q)], right)
        b2q0 = copy(6, comm_ref.at[3, pl.ds(0, h_q)],
                    comm_ref.at[5, pl.ds(0, h_q)], left)
        b2q1 = copy(7, comm_ref.at[3, pl.ds(h_q, h_q)],
                    comm_ref.at[5, pl.ds(h_q, h_q)], left)

        def gemm_block(row_start, src):
            out_ref[pl.ds(row_start, src.shape[0]), :] = jnp.maximum(
                jnp.dot(src, w_ref[:, :], preferred_element_type=jnp.float32),
                0.0,
            )

        gemm_block(my_pos * m_per, x_ref[:, :])

        barrier_sem = pltpu.get_barrier_semaphore()
        for nbr in [left, right]:
            pl.semaphore_signal(
                barrier_sem, inc=1,
                device_id=(nbr,), device_id_type=pl.DeviceIdType.MESH,
            )
        pl.semaphore_wait(barrier_sem, 2)

        f1a.start()
        b1b.start()
        f1b.start()
        b1a.start()

        f1a.wait_recv()
        f2q0.start()
        f2q1.start()
        b1b.wait_recv()
        b2q0.start()
        b2q1.start()

        gemm_block(left * m_per, comm_ref[0, :, :])
        f1b.wait_recv()
        gemm_block(left * m_per + h_half, comm_ref[1, :, :])
        gemm_block(right * m_per + h_half, comm_ref[3, :, :])
        b1a.wait_recv()
        gemm_block(right * m_per, comm_ref[2, :, :])
        f2q0.wait_recv()
        gemm_block(opp * m_per, comm_ref[4, pl.ds(0, h_q), :])
        b2q0.wait_recv()
        gemm_block(opp * m_per + h_half, comm_ref[5, pl.ds(0, h_q), :])
        f2q1.wait_recv()
        gemm_block(opp * m_per + h_q, comm_ref[4, pl.ds(h_q, h_q), :])
        b2q1.wait_recv()
        gemm_block(opp * m_per + h_half + h_q,
                   comm_ref[5, pl.ds(h_q, h_q), :])

        for d in [f1a, f1b, b1b, b1a, f2q0, f2q1, b2q0, b2q1]:
            d.wait_send()

    return pl.pallas_call(
        body,
        out_shape=jax.ShapeDtypeStruct((N_DEV * m_per, n), jnp.float32),
        in_specs=[
            pl.BlockSpec(memory_space=pltpu.VMEM),
            pl.BlockSpec(memory_space=pltpu.VMEM),
        ],
        out_specs=pl.BlockSpec(memory_space=pltpu.VMEM),
        scratch_shapes=[
            pltpu.VMEM((6, h_half, k), jnp.float32),
            pltpu.SemaphoreType.DMA((8,)),
            pltpu.SemaphoreType.DMA((8,)),
        ],
        compiler_params=pltpu.CompilerParams(collective_id=0),
    )(x, w_mat)


# device time: 25771 ns/iter; 1.0197x vs baseline; 1.0197x over previous
import jax
import jax.numpy as jnp
from jax import lax
from jax.experimental import pallas as pl
from jax.experimental.pallas import tpu as pltpu

N_DEV = 4


def kernel(x, w_mat):
    m_per, k = x.shape
    n = w_mat.shape[1]
    h_half = m_per // 2
    h_q = h_half // 2

    def body(x_ref, w_ref, out_ref, comm_ref, send_sems, recv_sems):
        my_pos = lax.axis_index("i")
        left = (my_pos - 1) % N_DEV
        right = (my_pos + 1) % N_DEV
        opp = (my_pos + 2) % N_DEV

        def copy(t, src, dst, target):
            return pltpu.make_async_remote_copy(
                src_ref=src,
                dst_ref=dst,
                send_sem=send_sems.at[t],
                recv_sem=recv_sems.at[t],
                device_id=(target,),
                device_id_type=pl.DeviceIdType.MESH,
            )

        x_a = x_ref.at[pl.ds(0, h_half)]
        x_b = x_ref.at[pl.ds(h_half, h_half)]
        f1a = copy(0, x_a, comm_ref.at[0], right)
        f1b = copy(1, x_b, comm_ref.at[1], right)
        b1b = copy(2, x_b, comm_ref.at[3], left)
        b1a = copy(3, x_a, comm_ref.at[2], left)
        f2q0 = copy(4, comm_ref.at[0, pl.ds(0, h_q)],
                    comm_ref.at[4, pl.ds(0, h_q)], right)
        f2q1 = copy(5, comm_ref.at[0, pl.ds(h_q, h_q)],
                    comm_ref.at[4, pl.ds(h_q, h_q)], right)
        b2q0 = copy(6, comm_ref.at[3, pl.ds(0, h_q)],
                    comm_ref.at[5, pl.ds(0, h_q)], left)
        b2q1 = copy(7, comm_ref.at[3, pl.ds(h_q, h_q)],
                    comm_ref.at[5, pl.ds(h_q, h_q)], left)

        def gemm_block(row_start, src):
            out_ref[pl.ds(row_start, src.shape[0]), :] = jnp.maximum(
                jnp.dot(src, w_ref[:, :], preferred_element_type=jnp.float32),
                0.0,
            )

        barrier_sem = pltpu.get_barrier_semaphore()
        for nbr in [left, right]:
            pl.semaphore_signal(
                barrier_sem, inc=1,
                device_id=(nbr,), device_id_type=pl.DeviceIdType.MESH,
            )
        gemm_block(my_pos * m_per, x_ref[:, :])
        pl.semaphore_wait(barrier_sem, 2)

        f1a.start()
        b1b.start()
        f1b.start()
        b1a.start()

        f1a.wait_recv()
        f2q0.start()
        f2q1.start()
        b1b.wait_recv()
        b2q0.start()
        b2q1.start()

        gemm_block(left * m_per, comm_ref[0, :, :])
        f1b.wait_recv()
        gemm_block(left * m_per + h_half, comm_ref[1, :, :])
        gemm_block(right * m_per + h_half, comm_ref[3, :, :])
        b1a.wait_recv()
        gemm_block(right * m_per, comm_ref[2, :, :])
        f2q0.wait_recv()
        gemm_block(opp * m_per, comm_ref[4, pl.ds(0, h_q), :])
        b2q0.wait_recv()
        gemm_block(opp * m_per + h_half, comm_ref[5, pl.ds(0, h_q), :])
        f2q1.wait_recv()
        gemm_block(opp * m_per + h_q, comm_ref[4, pl.ds(h_q, h_q), :])
        b2q1.wait_recv()
        gemm_block(opp * m_per + h_half + h_q,
                   comm_ref[5, pl.ds(h_q, h_q), :])

        for d in [f1a, f1b, b1b, b1a, f2q0, f2q1, b2q0, b2q1]:
            d.wait_send()

    return pl.pallas_call(
        body,
        out_shape=jax.ShapeDtypeStruct((N_DEV * m_per, n), jnp.float32),
        in_specs=[
            pl.BlockSpec(memory_space=pltpu.VMEM),
            pl.BlockSpec(memory_space=pltpu.VMEM),
        ],
        out_specs=pl.BlockSpec(memory_space=pltpu.VMEM),
        scratch_shapes=[
            pltpu.VMEM((6, h_half, k), jnp.float32),
            pltpu.SemaphoreType.DMA((8,)),
            pltpu.SemaphoreType.DMA((8,)),
        ],
        compiler_params=pltpu.CompilerParams(collective_id=0),
    )(x, w_mat)


# device time: 16743 ns/iter; 1.5696x vs baseline; 1.5392x over previous
import jax
import jax.numpy as jnp
from jax import lax
from jax.experimental import pallas as pl
from jax.experimental.pallas import tpu as pltpu

N_DEV = 4


def kernel(x, w_mat):
    m_per, k = x.shape
    n = w_mat.shape[1]
    h_half = m_per // 2
    h_q = h_half // 2

    def body(x_ref, w_ref, out_ref, xb_ref, wb_ref, comm_ref,
             send_sems, recv_sems):
        my_pos = lax.axis_index("i")
        left = (my_pos - 1) % N_DEV
        right = (my_pos + 1) % N_DEV
        opp = (my_pos + 2) % N_DEV

        def copy(t, src, dst, target):
            return pltpu.make_async_remote_copy(
                src_ref=src,
                dst_ref=dst,
                send_sem=send_sems.at[t],
                recv_sem=recv_sems.at[t],
                device_id=(target,),
                device_id_type=pl.DeviceIdType.MESH,
            )

        x_a = xb_ref.at[pl.ds(0, h_half)]
        x_b = xb_ref.at[pl.ds(h_half, h_half)]
        f1a = copy(0, x_a, comm_ref.at[0], right)
        f1b = copy(1, x_b, comm_ref.at[1], right)
        b1b = copy(2, x_b, comm_ref.at[3], left)
        b1a = copy(3, x_a, comm_ref.at[2], left)
        f2q0 = copy(4, comm_ref.at[0, pl.ds(0, h_q)],
                    comm_ref.at[4, pl.ds(0, h_q)], right)
        f2q1 = copy(5, comm_ref.at[0, pl.ds(h_q, h_q)],
                    comm_ref.at[4, pl.ds(h_q, h_q)], right)
        b2q0 = copy(6, comm_ref.at[3, pl.ds(0, h_q)],
                    comm_ref.at[5, pl.ds(0, h_q)], left)
        b2q1 = copy(7, comm_ref.at[3, pl.ds(h_q, h_q)],
                    comm_ref.at[5, pl.ds(h_q, h_q)], left)

        def gemm_block(row_start, src):
            out_ref[pl.ds(row_start, src.shape[0]), :] = jnp.maximum(
                jnp.dot(src, wb_ref[:, :],
                        preferred_element_type=jnp.float32),
                0.0,
            )

        barrier_sem = pltpu.get_barrier_semaphore()
        for nbr in [left, right]:
            pl.semaphore_signal(
                barrier_sem, inc=1,
                device_id=(nbr,), device_id_type=pl.DeviceIdType.MESH,
            )
        xb_ref[:, :] = x_ref[:, :].astype(jnp.bfloat16)
        wb_ref[:, :] = w_ref[:, :].astype(jnp.bfloat16)
        gemm_block(my_pos * m_per, xb_ref[:, :])
        pl.semaphore_wait(barrier_sem, 2)

        f1a.start()
        b1b.start()
        f1b.start()
        b1a.start()

        f1a.wait_recv()
        f2q0.start()
        f2q1.start()
        b1b.wait_recv()
        b2q0.start()
        b2q1.start()

        gemm_block(left * m_per, comm_ref[0, :, :])
        f1b.wait_recv()
        gemm_block(left * m_per + h_half, comm_ref[1, :, :])
        gemm_block(right * m_per + h_half, comm_ref[3, :, :])
        b1a.wait_recv()
        gemm_block(right * m_per, comm_ref[2, :, :])
        f2q0.wait_recv()
        gemm_block(opp * m_per, comm_ref[4, pl.ds(0, h_q), :])
        b2q0.wait_recv()
        gemm_block(opp * m_per + h_half, comm_ref[5, pl.ds(0, h_q), :])
        f2q1.wait_recv()
        gemm_block(opp * m_per + h_q, comm_ref[4, pl.ds(h_q, h_q), :])
        b2q1.wait_recv()
        gemm_block(opp * m_per + h_half + h_q,
                   comm_ref[5, pl.ds(h_q, h_q), :])

        for d in [f1a, f1b, b1b, b1a, f2q0, f2q1, b2q0, b2q1]:
            d.wait_send()

    return pl.pallas_call(
        body,
        out_shape=jax.ShapeDtypeStruct((N_DEV * m_per, n), jnp.float32),
        in_specs=[
            pl.BlockSpec(memory_space=pltpu.VMEM),
            pl.BlockSpec(memory_space=pltpu.VMEM),
        ],
        out_specs=pl.BlockSpec(memory_space=pltpu.VMEM),
        scratch_shapes=[
            pltpu.VMEM((m_per, k), jnp.bfloat16),
            pltpu.VMEM((k, n), jnp.bfloat16),
            pltpu.VMEM((6, h_half, k), jnp.bfloat16),
            pltpu.SemaphoreType.DMA((8,)),
            pltpu.SemaphoreType.DMA((8,)),
        ],
        compiler_params=pltpu.CompilerParams(collective_id=0),
    )(x, w_mat)
